# baseline (speedup 1.0000x reference)
import dataclasses
import functools

import jax
import jax.numpy as jnp
from jax import lax
from jax.experimental import pallas as pl
from jax.experimental.pallas import tpu as pltpu
from jax.experimental.pallas import tpu_sc as plsc

TOTAL_TOK = 32768
BATCH = 16
D = 1024
NC = 2
NS = 16
COLS = D // NC
CH = 64
NG = COLS // 16


def _sc_segment_mean(values, starts, ends):
    mesh = plsc.VectorSubcoreMesh(core_axis_name="c", subcore_axis_name="s")
    cp = pltpu.CompilerParams()
    if "needs_layout_passes" in pltpu.CompilerParams.__dataclass_fields__:
        cp = dataclasses.replace(cp, needs_layout_passes=False)

    @functools.partial(
        pl.kernel,
        mesh=mesh,
        out_type=jax.ShapeDtypeStruct((BATCH, D), jnp.float32),
        scratch_types=[
            pltpu.VMEM((BATCH,), jnp.int32),
            pltpu.VMEM((BATCH,), jnp.int32),
            pltpu.VMEM((CH, COLS), jnp.float32),
            pltpu.VMEM((CH, COLS), jnp.float32),
            pltpu.VMEM((BATCH, COLS), jnp.float32),
            pltpu.VMEM((BATCH, COLS), jnp.float32),
            pltpu.VMEM_SHARED((NS, BATCH, COLS), jnp.float32),
            pltpu.SemaphoreType.DMA,
            pltpu.SemaphoreType.DMA,
        ],
        compiler_params=cp,
    )
    def run(values_hbm, starts_hbm, ends_hbm, out_hbm, sv_ref, ev_ref, buf0,
            buf1, accl, tmp, shared, gsem0, gsem1):
        c = lax.axis_index("c")
        s = lax.axis_index("s")
        col0 = c * COLS

        pltpu.sync_copy(starts_hbm, sv_ref)
        pltpu.sync_copy(ends_hbm, ev_ref)
        sv = sv_ref[...]
        ev = ev_ref[...]
        lens = ev - sv
        cum = plsc.cumsum(lens)
        total = jnp.sum(lens)

        lo = lax.shift_right_logical(s * total, 4)
        hi = lax.shift_right_logical((s + 1) * total, 4)

        zeros16 = jnp.zeros((16,), jnp.float32)
        for b in range(BATCH):
            for j in range(NG):
                accl[b, pl.ds(j * 16, 16)] = zeros16

        lane = lax.broadcasted_iota(jnp.int32, (16,), 0)
        zero_i = jnp.zeros((16,), jnp.int32)

        @pl.loop(0, BATCH)
        def _(b):
            sel = lane == b
            len_b = jnp.sum(jnp.where(sel, lens, zero_i))
            seg_hi = jnp.sum(jnp.where(sel, cum, zero_i))
            seg_lo = seg_hi - len_b
            start_b = jnp.sum(jnp.where(sel, sv, zero_i))
            ov_lo = jnp.maximum(seg_lo, lo)
            ov_hi = jnp.minimum(seg_hi, hi)
            n = ov_hi - ov_lo
            r0 = start_b + (ov_lo - seg_lo)
            ar0 = lax.shift_left(lax.shift_right_logical(r0, 3), 3)
            span = jnp.maximum(r0 - ar0 + n, 0)
            nch = lax.shift_right_logical(span + (CH - 1), 6)
            npairs = lax.shift_right_logical(nch + 1, 1)

            def gather(k, buf, sem):
                base0 = ar0 + k * CH
                base = pl.multiple_of(
                    jnp.minimum(base0, TOTAL_TOK - CH), 8)
                pltpu.async_copy(
                    values_hbm.at[pl.ds(base, CH), pl.ds(col0, COLS)], buf,
                    sem)

            def wait(buf, sem):
                pltpu.make_async_copy(
                    values_hbm.at[pl.ds(0, CH), pl.ds(col0, COLS)], buf,
                    sem).wait()

            def accum(k, buf):
                base0 = ar0 + k * CH
                base = jnp.minimum(base0, TOTAL_TOK - CH)
                mlo = jnp.maximum(r0, base0) - base
                mhi = jnp.minimum(r0 + n, base0 + CH) - base

                def row(i, carry):
                    for j in range(NG):
                        plsc.addupdate(accl.at[b, pl.ds(j * 16, 16)],
                                       buf[i, pl.ds(j * 16, 16)])
                    return carry

                lax.fori_loop(mlo, jnp.maximum(mhi, mlo), row, 0)

            @pl.when(n > 0)
            def _():
                gather(0, buf0, gsem0)

                def pair(p, carry):
                    k0 = 2 * p
                    gather(k0 + 1, buf1, gsem1)
                    wait(buf0, gsem0)
                    accum(k0, buf0)
                    gather(k0 + 2, buf0, gsem0)
                    wait(buf1, gsem1)
                    accum(k0 + 1, buf1)
                    return carry

                lax.fori_loop(0, npairs, pair, 0)
                wait(buf0, gsem0)

        pltpu.sync_copy(accl, shared.at[s])

        for r in (1, 2, 4, 8):
            plsc.subcore_barrier()

            @pl.when((s & (2 * r - 1)) == 0)
            def _(r=r):
                pltpu.sync_copy(shared.at[s + r], tmp)
                for b in range(BATCH):
                    for j in range(NG):
                        sl = (b, pl.ds(j * 16, 16))
                        accl[sl] = accl[sl] + tmp[sl]
                if r < 8:
                    pltpu.sync_copy(accl, shared.at[s])

        @pl.when(s == 0)
        def _():
            invv = jnp.ones((16,), jnp.float32) / lens.astype(jnp.float32)
            for b in range(BATCH):
                sel_b = lane == b
                inv_b = jnp.sum(jnp.where(sel_b, invv, zeros16))
                scale = lax.broadcast_in_dim(inv_b, (16,), ())
                for j in range(NG):
                    tmp[b, pl.ds(j * 16, 16)] = (
                        accl[b, pl.ds(j * 16, 16)] * scale)
            pltpu.sync_copy(tmp, out_hbm.at[:, pl.ds(col0, COLS)])

    return run(values, starts, ends)


def kernel(values, indices):
    idx32 = indices.astype(jnp.int32)
    starts = idx32[:, 0]
    ends = idx32[:, 1]
    return _sc_segment_mean(values, starts, ends)


# speedup vs baseline: 1.7274x; 1.7274x over previous
import dataclasses
import functools

import jax
import jax.numpy as jnp
from jax import lax
from jax.experimental import pallas as pl
from jax.experimental.pallas import tpu as pltpu
from jax.experimental.pallas import tpu_sc as plsc

TOTAL_TOK = 32768
BATCH = 16
D = 1024
NC = 2
NS = 16
COLS = D // NC
CH = 64
NG = COLS // 16


def _sc_segment_mean(values, starts, ends):
    mesh = plsc.VectorSubcoreMesh(core_axis_name="c", subcore_axis_name="s")
    cp = pltpu.CompilerParams()
    if "needs_layout_passes" in pltpu.CompilerParams.__dataclass_fields__:
        cp = dataclasses.replace(cp, needs_layout_passes=False)

    @functools.partial(
        pl.kernel,
        mesh=mesh,
        out_type=jax.ShapeDtypeStruct((BATCH, D), jnp.float32),
        scratch_types=[
            pltpu.VMEM((BATCH,), jnp.int32),
            pltpu.VMEM((BATCH,), jnp.int32),
            pltpu.VMEM((CH, COLS), jnp.float32),
            pltpu.VMEM((CH, COLS), jnp.float32),
            pltpu.VMEM((BATCH, COLS), jnp.float32),
            pltpu.VMEM((BATCH, COLS), jnp.float32),
            pltpu.VMEM_SHARED((NS, BATCH, COLS), jnp.float32),
            pltpu.SemaphoreType.DMA,
            pltpu.SemaphoreType.DMA,
        ],
        compiler_params=cp,
    )
    def run(values_hbm, starts_hbm, ends_hbm, out_hbm, sv_ref, ev_ref, buf0,
            buf1, accl, tmp, shared, gsem0, gsem1):
        c = lax.axis_index("c")
        s = lax.axis_index("s")
        col0 = c * COLS

        pltpu.sync_copy(starts_hbm, sv_ref)
        pltpu.sync_copy(ends_hbm, ev_ref)
        sv = sv_ref[...]
        ev = ev_ref[...]
        lens = ev - sv
        cum = plsc.cumsum(lens)
        total = jnp.sum(lens)

        lo = lax.shift_right_logical(s * total, 4)
        hi = lax.shift_right_logical((s + 1) * total, 4)

        zeros16 = jnp.zeros((16,), jnp.float32)
        for b in range(BATCH):
            for j in range(NG):
                accl[b, pl.ds(j * 16, 16)] = zeros16

        lane = lax.broadcasted_iota(jnp.int32, (16,), 0)
        zero_i = jnp.zeros((16,), jnp.int32)

        @pl.loop(0, BATCH)
        def _(b):
            sel = lane == b
            len_b = jnp.sum(jnp.where(sel, lens, zero_i))
            seg_hi = jnp.sum(jnp.where(sel, cum, zero_i))
            seg_lo = seg_hi - len_b
            start_b = jnp.sum(jnp.where(sel, sv, zero_i))
            ov_lo = jnp.maximum(seg_lo, lo)
            ov_hi = jnp.minimum(seg_hi, hi)
            n = ov_hi - ov_lo
            r0 = start_b + (ov_lo - seg_lo)
            ar0 = lax.shift_left(lax.shift_right_logical(r0, 3), 3)
            span = jnp.maximum(r0 - ar0 + n, 0)
            nch = lax.shift_right_logical(span + (CH - 1), 6)
            npairs = lax.shift_right_logical(nch + 1, 1)

            def gather(k, buf, sem):
                base0 = ar0 + k * CH
                base = pl.multiple_of(
                    jnp.minimum(base0, TOTAL_TOK - CH), 8)
                pltpu.async_copy(
                    values_hbm.at[pl.ds(base, CH), pl.ds(col0, COLS)], buf,
                    sem)

            def wait(buf, sem):
                pltpu.make_async_copy(
                    values_hbm.at[pl.ds(0, CH), pl.ds(col0, COLS)], buf,
                    sem).wait()

            def accum(k, buf, acc32):
                base0 = ar0 + k * CH
                base = jnp.minimum(base0, TOTAL_TOK - CH)
                mlo = jnp.maximum(r0, base0) - base
                mhi = jnp.minimum(r0 + n, base0 + CH) - base

                def row(i, a):
                    return tuple(
                        a[j] + buf[i, pl.ds(j * 16, 16)] for j in range(NG))

                return plsc.parallel_loop(
                    mlo, jnp.maximum(mhi, mlo), unroll=4, carry=acc32)(row)

            @pl.when(n > 0)
            def _():
                gather(0, buf0, gsem0)
                acc0 = tuple(zeros16 for _ in range(NG))

                def pair(p, acc32):
                    k0 = 2 * p
                    gather(k0 + 1, buf1, gsem1)
                    wait(buf0, gsem0)
                    acc32 = accum(k0, buf0, acc32)
                    gather(k0 + 2, buf0, gsem0)
                    wait(buf1, gsem1)
                    acc32 = accum(k0 + 1, buf1, acc32)
                    return acc32

                acc32 = lax.fori_loop(0, npairs, pair, acc0)
                wait(buf0, gsem0)
                for j in range(NG):
                    accl[b, pl.ds(j * 16, 16)] = acc32[j]

        pltpu.sync_copy(accl, shared.at[s])

        for r in (1, 2, 4, 8):
            plsc.subcore_barrier()

            @pl.when((s & (2 * r - 1)) == 0)
            def _(r=r):
                pltpu.sync_copy(shared.at[s + r], tmp)
                for b in range(BATCH):
                    for j in range(NG):
                        sl = (b, pl.ds(j * 16, 16))
                        accl[sl] = accl[sl] + tmp[sl]
                if r < 8:
                    pltpu.sync_copy(accl, shared.at[s])

        @pl.when(s == 0)
        def _():
            invv = jnp.ones((16,), jnp.float32) / lens.astype(jnp.float32)
            for b in range(BATCH):
                sel_b = lane == b
                inv_b = jnp.sum(jnp.where(sel_b, invv, zeros16))
                scale = lax.broadcast_in_dim(inv_b, (16,), ())
                for j in range(NG):
                    tmp[b, pl.ds(j * 16, 16)] = (
                        accl[b, pl.ds(j * 16, 16)] * scale)
            pltpu.sync_copy(tmp, out_hbm.at[:, pl.ds(col0, COLS)])

    return run(values, starts, ends)


def kernel(values, indices):
    idx32 = indices.astype(jnp.int32)
    starts = idx32[:, 0]
    ends = idx32[:, 1]
    return _sc_segment_mean(values, starts, ends)


# speedup vs baseline: 1.9157x; 1.1091x over previous
import dataclasses
import functools

import jax
import jax.numpy as jnp
from jax import lax
from jax.experimental import pallas as pl
from jax.experimental.pallas import tpu as pltpu
from jax.experimental.pallas import tpu_sc as plsc

TOTAL_TOK = 32768
BATCH = 16
D = 1024
NC = 2
NS = 16
COLS = D // NC
CH = 64
NG = COLS // 16


def _sc_segment_mean(values, starts, ends):
    mesh = plsc.VectorSubcoreMesh(core_axis_name="c", subcore_axis_name="s")
    cp = pltpu.CompilerParams()
    if "needs_layout_passes" in pltpu.CompilerParams.__dataclass_fields__:
        cp = dataclasses.replace(cp, needs_layout_passes=False)

    @functools.partial(
        pl.kernel,
        mesh=mesh,
        out_type=jax.ShapeDtypeStruct((BATCH, D), jnp.float32),
        scratch_types=[
            pltpu.VMEM((BATCH,), jnp.int32),
            pltpu.VMEM((BATCH,), jnp.int32),
            pltpu.VMEM((CH, COLS), jnp.float32),
            pltpu.VMEM((CH, COLS), jnp.float32),
            pltpu.VMEM((BATCH, COLS), jnp.float32),
            pltpu.VMEM((BATCH, COLS), jnp.float32),
            pltpu.VMEM_SHARED((NS, BATCH, COLS), jnp.float32),
            pltpu.SemaphoreType.DMA,
            pltpu.SemaphoreType.DMA,
        ],
        compiler_params=cp,
    )
    def run(values_hbm, starts_hbm, ends_hbm, out_hbm, sv_ref, ev_ref, buf0,
            buf1, accl, tmp, shared, gsem0, gsem1):
        c = lax.axis_index("c")
        s = lax.axis_index("s")
        col0 = c * COLS

        pltpu.sync_copy(starts_hbm, sv_ref)
        pltpu.sync_copy(ends_hbm, ev_ref)
        sv = sv_ref[...]
        ev = ev_ref[...]
        lens = ev - sv
        cum = plsc.cumsum(lens)
        total = jnp.sum(lens)

        lo = lax.shift_right_logical(s * total, 4)
        hi = lax.shift_right_logical((s + 1) * total, 4)

        zeros16 = jnp.zeros((16,), jnp.float32)
        for b in range(BATCH):
            for j in range(NG):
                accl[b, pl.ds(j * 16, 16)] = zeros16

        lane = lax.broadcasted_iota(jnp.int32, (16,), 0)
        zero_i = jnp.zeros((16,), jnp.int32)

        @pl.loop(0, BATCH)
        def _(b):
            sel = lane == b
            len_b = jnp.sum(jnp.where(sel, lens, zero_i))
            seg_hi = jnp.sum(jnp.where(sel, cum, zero_i))
            seg_lo = seg_hi - len_b
            start_b = jnp.sum(jnp.where(sel, sv, zero_i))
            ov_lo = jnp.maximum(seg_lo, lo)
            ov_hi = jnp.minimum(seg_hi, hi)
            n = ov_hi - ov_lo
            r0 = start_b + (ov_lo - seg_lo)
            ar0 = lax.shift_left(lax.shift_right_logical(r0, 3), 3)
            span = jnp.maximum(r0 - ar0 + n, 0)
            nch = lax.shift_right_logical(span + (CH - 1), 6)
            npairs = lax.shift_right_logical(nch + 1, 1)

            def gather(k, buf, sem):
                base0 = ar0 + k * CH
                base = pl.multiple_of(
                    jnp.minimum(base0, TOTAL_TOK - CH), 8)
                pltpu.async_copy(
                    values_hbm.at[pl.ds(base, CH), pl.ds(col0, COLS)], buf,
                    sem)

            def wait(buf, sem):
                pltpu.make_async_copy(
                    values_hbm.at[pl.ds(0, CH), pl.ds(col0, COLS)], buf,
                    sem).wait()

            def accum(k, buf, acc32):
                base0 = ar0 + k * CH
                base = jnp.minimum(base0, TOTAL_TOK - CH)
                mlo = jnp.maximum(r0, base0) - base
                mhi = jnp.minimum(r0 + n, base0 + CH) - base

                def row(i, a):
                    return tuple(
                        a[j] + buf[i, pl.ds(j * 16, 16)] for j in range(NG))

                return plsc.parallel_loop(
                    mlo, jnp.maximum(mhi, mlo), unroll=2, carry=acc32)(row)

            @pl.when(n > 0)
            def _():
                gather(0, buf0, gsem0)
                acc0 = tuple(zeros16 for _ in range(NG))

                def pair(p, acc32):
                    k0 = 2 * p
                    gather(k0 + 1, buf1, gsem1)
                    wait(buf0, gsem0)
                    acc32 = accum(k0, buf0, acc32)
                    gather(k0 + 2, buf0, gsem0)
                    wait(buf1, gsem1)
                    acc32 = accum(k0 + 1, buf1, acc32)
                    return acc32

                acc32 = lax.fori_loop(0, npairs, pair, acc0)
                wait(buf0, gsem0)
                for j in range(NG):
                    accl[b, pl.ds(j * 16, 16)] = acc32[j]

        pltpu.sync_copy(accl, shared.at[s])

        for r in (1, 2, 4, 8):
            plsc.subcore_barrier()

            @pl.when((s & (2 * r - 1)) == 0)
            def _(r=r):
                pltpu.sync_copy(shared.at[s + r], tmp)
                for b in range(BATCH):
                    for j in range(NG):
                        sl = (b, pl.ds(j * 16, 16))
                        accl[sl] = accl[sl] + tmp[sl]
                if r < 8:
                    pltpu.sync_copy(accl, shared.at[s])

        @pl.when(s == 0)
        def _():
            invv = jnp.ones((16,), jnp.float32) / lens.astype(jnp.float32)
            for b in range(BATCH):
                sel_b = lane == b
                inv_b = jnp.sum(jnp.where(sel_b, invv, zeros16))
                scale = lax.broadcast_in_dim(inv_b, (16,), ())
                for j in range(NG):
                    tmp[b, pl.ds(j * 16, 16)] = (
                        accl[b, pl.ds(j * 16, 16)] * scale)
            pltpu.sync_copy(tmp, out_hbm.at[:, pl.ds(col0, COLS)])

    return run(values, starts, ends)


def kernel(values, indices):
    idx32 = indices.astype(jnp.int32)
    starts = idx32[:, 0]
    ends = idx32[:, 1]
    return _sc_segment_mean(values, starts, ends)


# speedup vs baseline: 2.0599x; 1.0753x over previous
import dataclasses
import functools

import jax
import jax.numpy as jnp
from jax import lax
from jax.experimental import pallas as pl
from jax.experimental.pallas import tpu as pltpu
from jax.experimental.pallas import tpu_sc as plsc

TOTAL_TOK = 32768
BATCH = 16
D = 1024
NC = 2
NS = 16
COLS = D // NC
CH = 64
NG = COLS // 16


def _sc_segment_mean(values, starts, ends):
    mesh = plsc.VectorSubcoreMesh(core_axis_name="c", subcore_axis_name="s")
    cp = pltpu.CompilerParams()
    if "needs_layout_passes" in pltpu.CompilerParams.__dataclass_fields__:
        cp = dataclasses.replace(cp, needs_layout_passes=False)

    @functools.partial(
        pl.kernel,
        mesh=mesh,
        out_type=jax.ShapeDtypeStruct((BATCH, D), jnp.float32),
        scratch_types=[
            pltpu.VMEM((BATCH,), jnp.int32),
            pltpu.VMEM((BATCH,), jnp.int32),
            pltpu.VMEM((CH, COLS), jnp.float32),
            pltpu.VMEM((CH, COLS), jnp.float32),
            pltpu.VMEM((BATCH, COLS), jnp.float32),
            pltpu.VMEM((BATCH, COLS), jnp.float32),
            pltpu.VMEM_SHARED((NS, BATCH, COLS), jnp.float32),
            pltpu.SemaphoreType.DMA,
            pltpu.SemaphoreType.DMA,
        ],
        compiler_params=cp,
    )
    def run(values_hbm, starts_hbm, ends_hbm, out_hbm, sv_ref, ev_ref, buf0,
            buf1, accl, tmp, shared, gsem0, gsem1):
        c = lax.axis_index("c")
        s = lax.axis_index("s")
        col0 = c * COLS

        pltpu.sync_copy(starts_hbm, sv_ref)
        pltpu.sync_copy(ends_hbm, ev_ref)
        sv = sv_ref[...]
        ev = ev_ref[...]
        lens = ev - sv
        cum = plsc.cumsum(lens)
        total = jnp.sum(lens)

        lo = lax.shift_right_logical(s * total, 4)
        hi = lax.shift_right_logical((s + 1) * total, 4)

        zeros16 = jnp.zeros((16,), jnp.float32)
        for b in range(BATCH):
            for j in range(NG):
                accl[b, pl.ds(j * 16, 16)] = zeros16

        lane = lax.broadcasted_iota(jnp.int32, (16,), 0)
        zero_i = jnp.zeros((16,), jnp.int32)

        @pl.loop(0, BATCH)
        def _(b):
            sel = lane == b
            len_b = jnp.sum(jnp.where(sel, lens, zero_i))
            seg_hi = jnp.sum(jnp.where(sel, cum, zero_i))
            seg_lo = seg_hi - len_b
            start_b = jnp.sum(jnp.where(sel, sv, zero_i))
            ov_lo = jnp.maximum(seg_lo, lo)
            ov_hi = jnp.minimum(seg_hi, hi)
            n = ov_hi - ov_lo
            r0 = start_b + (ov_lo - seg_lo)
            ar0 = lax.shift_left(lax.shift_right_logical(r0, 3), 3)
            span = jnp.maximum(r0 - ar0 + n, 0)
            nch = lax.shift_right_logical(span + (CH - 1), 6)
            npairs = lax.shift_right_logical(nch + 1, 1)

            def gather(k, buf, sem):
                base0 = ar0 + k * CH
                base = pl.multiple_of(
                    jnp.minimum(base0, TOTAL_TOK - CH), 8)
                pltpu.async_copy(
                    values_hbm.at[pl.ds(base, CH), pl.ds(col0, COLS)], buf,
                    sem)

            def wait(buf, sem):
                pltpu.make_async_copy(
                    values_hbm.at[pl.ds(0, CH), pl.ds(col0, COLS)], buf,
                    sem).wait()

            def accum(k, buf, acc32):
                base0 = ar0 + k * CH
                base = jnp.minimum(base0, TOTAL_TOK - CH)
                mlo = jnp.maximum(r0, base0) - base
                mhi = jnp.minimum(r0 + n, base0 + CH) - base

                def row(i, a):
                    return tuple(
                        a[j] + buf[i, pl.ds(j * 16, 16)] for j in range(NG))

                return lax.fori_loop(mlo, jnp.maximum(mhi, mlo), row, acc32)

            @pl.when(n > 0)
            def _():
                gather(0, buf0, gsem0)
                acc0 = tuple(zeros16 for _ in range(NG))

                def pair(p, acc32):
                    k0 = 2 * p
                    has1 = k0 + 1 < nch
                    has2 = k0 + 2 < nch

                    @pl.when(has1)
                    def _():
                        gather(k0 + 1, buf1, gsem1)

                    wait(buf0, gsem0)
                    acc32 = accum(k0, buf0, acc32)

                    @pl.when(has2)
                    def _():
                        gather(k0 + 2, buf0, gsem0)

                    acc_ref = acc32

                    def tail(a):
                        wait(buf1, gsem1)
                        return accum(k0 + 1, buf1, a)

                    return lax.cond(has1, tail, lambda a: a, acc_ref)

                acc32 = lax.fori_loop(0, npairs, pair, acc0)
                for j in range(NG):
                    accl[b, pl.ds(j * 16, 16)] = acc32[j]

        pltpu.sync_copy(accl, shared.at[s])

        for r in (1, 2, 4, 8):
            plsc.subcore_barrier()

            @pl.when((s & (2 * r - 1)) == 0)
            def _(r=r):
                pltpu.sync_copy(shared.at[s + r], tmp)
                for b in range(BATCH):
                    for j in range(NG):
                        sl = (b, pl.ds(j * 16, 16))
                        accl[sl] = accl[sl] + tmp[sl]
                if r < 8:
                    pltpu.sync_copy(accl, shared.at[s])

        @pl.when(s == 0)
        def _():
            invv = jnp.ones((16,), jnp.float32) / lens.astype(jnp.float32)
            for b in range(BATCH):
                sel_b = lane == b
                inv_b = jnp.sum(jnp.where(sel_b, invv, zeros16))
                scale = lax.broadcast_in_dim(inv_b, (16,), ())
                for j in range(NG):
                    tmp[b, pl.ds(j * 16, 16)] = (
                        accl[b, pl.ds(j * 16, 16)] * scale)
            pltpu.sync_copy(tmp, out_hbm.at[:, pl.ds(col0, COLS)])

    return run(values, starts, ends)


def kernel(values, indices):
    idx32 = indices.astype(jnp.int32)
    starts = idx32[:, 0]
    ends = idx32[:, 1]
    return _sc_segment_mean(values, starts, ends)


# speedup vs baseline: 2.5022x; 1.2147x over previous
import dataclasses
import functools

import jax
import jax.numpy as jnp
from jax import lax
from jax.experimental import pallas as pl
from jax.experimental.pallas import tpu as pltpu
from jax.experimental.pallas import tpu_sc as plsc

TOTAL_TOK = 32768
BATCH = 16
D = 1024
NC = 2
NS = 16
COLS = D // NC
CH = 64
NG = COLS // 16


def _sc_segment_mean(values, starts_ends):
    mesh = plsc.VectorSubcoreMesh(core_axis_name="c", subcore_axis_name="s")
    cp = pltpu.CompilerParams()
    if "needs_layout_passes" in pltpu.CompilerParams.__dataclass_fields__:
        cp = dataclasses.replace(cp, needs_layout_passes=False)

    @functools.partial(
        pl.kernel,
        mesh=mesh,
        out_type=jax.ShapeDtypeStruct((BATCH, D), jnp.float32),
        scratch_types=[
            pltpu.VMEM((2, BATCH), jnp.int32),
            pltpu.VMEM((CH, COLS), jnp.float32),
            pltpu.VMEM((CH, COLS), jnp.float32),
            pltpu.VMEM((BATCH, COLS), jnp.float32),
            pltpu.VMEM((BATCH, COLS), jnp.float32),
            pltpu.VMEM_SHARED((NS, BATCH, COLS), jnp.float32),
            pltpu.SemaphoreType.DMA,
            pltpu.SemaphoreType.DMA,
        ],
        compiler_params=cp,
    )
    def run(values_hbm, se_hbm, out_hbm, se_ref, buf0,
            buf1, accl, tmp, shared, gsem0, gsem1):
        c = lax.axis_index("c")
        s = lax.axis_index("s")
        col0 = c * COLS

        pltpu.sync_copy(se_hbm, se_ref)
        sv = se_ref[0]
        ev = se_ref[1]
        lens = ev - sv
        cum = plsc.cumsum(lens)
        total = jnp.sum(lens)

        lo = lax.shift_right_logical(s * total, 4)
        hi = lax.shift_right_logical((s + 1) * total, 4)

        zeros16 = jnp.zeros((16,), jnp.float32)

        @pl.loop(0, BATCH)
        def _(b):
            for j in range(NG):
                accl[b, pl.ds(j * 16, 16)] = zeros16

        lane = lax.broadcasted_iota(jnp.int32, (16,), 0)
        zero_i = jnp.zeros((16,), jnp.int32)

        @pl.loop(0, BATCH)
        def _(b):
            sel = lane == b
            len_b = jnp.sum(jnp.where(sel, lens, zero_i))
            seg_hi = jnp.sum(jnp.where(sel, cum, zero_i))
            seg_lo = seg_hi - len_b
            start_b = jnp.sum(jnp.where(sel, sv, zero_i))
            ov_lo = jnp.maximum(seg_lo, lo)
            ov_hi = jnp.minimum(seg_hi, hi)
            n = ov_hi - ov_lo
            r0 = start_b + (ov_lo - seg_lo)
            ar0 = lax.shift_left(lax.shift_right_logical(r0, 3), 3)
            span = jnp.maximum(r0 - ar0 + n, 0)
            nch = lax.shift_right_logical(span + (CH - 1), 6)
            npairs = lax.shift_right_logical(nch + 1, 1)

            def gather(k, buf, sem):
                base0 = ar0 + k * CH
                base = pl.multiple_of(
                    jnp.minimum(base0, TOTAL_TOK - CH), 8)
                pltpu.async_copy(
                    values_hbm.at[pl.ds(base, CH), pl.ds(col0, COLS)], buf,
                    sem)

            def wait(buf, sem):
                pltpu.make_async_copy(
                    values_hbm.at[pl.ds(0, CH), pl.ds(col0, COLS)], buf,
                    sem).wait()

            def accum(k, buf, acc32):
                base0 = ar0 + k * CH
                base = jnp.minimum(base0, TOTAL_TOK - CH)
                mlo = jnp.maximum(r0, base0) - base
                mhi = jnp.minimum(r0 + n, base0 + CH) - base

                def row(i, a):
                    return tuple(
                        a[j] + buf[i, pl.ds(j * 16, 16)] for j in range(NG))

                return lax.fori_loop(mlo, jnp.maximum(mhi, mlo), row, acc32)

            @pl.when(n > 0)
            def _():
                gather(0, buf0, gsem0)
                acc0 = tuple(zeros16 for _ in range(NG))

                def pair(p, acc32):
                    k0 = 2 * p
                    has1 = k0 + 1 < nch
                    has2 = k0 + 2 < nch

                    @pl.when(has1)
                    def _():
                        gather(k0 + 1, buf1, gsem1)

                    wait(buf0, gsem0)
                    acc32 = accum(k0, buf0, acc32)

                    @pl.when(has2)
                    def _():
                        gather(k0 + 2, buf0, gsem0)

                    acc_ref = acc32

                    def tail(a):
                        wait(buf1, gsem1)
                        return accum(k0 + 1, buf1, a)

                    return lax.cond(has1, tail, lambda a: a, acc_ref)

                acc32 = lax.fori_loop(0, npairs, pair, acc0)
                for j in range(NG):
                    accl[b, pl.ds(j * 16, 16)] = acc32[j]

        pltpu.sync_copy(accl, shared.at[s])

        def add_image(src):
            @pl.loop(0, BATCH)
            def _(b):
                for j in range(NG):
                    sl = (b, pl.ds(j * 16, 16))
                    accl[sl] = accl[sl] + src[sl]

        def combine3(p1, p2, p3):
            for p in (p1, p2, p3):
                pltpu.sync_copy(shared.at[p], tmp)
                add_image(tmp)

        plsc.subcore_barrier()

        @pl.when((s & 3) == 0)
        def _():
            combine3(s + 1, s + 2, s + 3)
            pltpu.sync_copy(accl, shared.at[s])

        plsc.subcore_barrier()

        @pl.when(s == 0)
        def _():
            combine3(4, 8, 12)

        @pl.when(s == 0)
        def _():
            invv = jnp.ones((16,), jnp.float32) / lens.astype(jnp.float32)

            @pl.loop(0, BATCH)
            def _(b):
                sel_b = lane == b
                inv_b = jnp.sum(jnp.where(sel_b, invv, zeros16))
                scale = lax.broadcast_in_dim(inv_b, (16,), ())
                for j in range(NG):
                    tmp[b, pl.ds(j * 16, 16)] = (
                        accl[b, pl.ds(j * 16, 16)] * scale)
            pltpu.sync_copy(tmp, out_hbm.at[:, pl.ds(col0, COLS)])

    return run(values, starts_ends)


def kernel(values, indices):
    idx32 = indices.astype(jnp.int32)
    return _sc_segment_mean(values, idx32.T)
